# baseline (speedup 1.0000x reference)
import jax
import jax.numpy as jnp
from jax.experimental import pallas as pl
from jax.experimental.pallas import tpu as pltpu


def _pick_blk(n):
    for b in (400, 200, 100, 50, 25):
        if n % b == 0:
            return b
    return n


def _layer1_body(adjf_ref, feat_ref, W1_ref, b1_ref, adjq_ref, x1_ref,
                 x1max_ref, featq_ref, fscale_ref):
    j = pl.program_id(0)
    n = adjf_ref.shape[1]

    @pl.when(j == 0)
    def _quant_feat():
        f = feat_ref[...]
        fm = jnp.maximum(jnp.max(jnp.abs(f)), 1e-30)
        fscale_ref[0] = fm
        scaled = f * (127.0 / fm)
        featq_ref[...] = (scaled + jnp.where(scaled >= 0.0, 0.5, -0.5)
                          ).astype(jnp.int8)

    q = (adjf_ref[...] * (127.0 * n) + 0.5).astype(jnp.int8)
    adjq_ref[...] = q
    z32 = jax.lax.dot_general(q, featq_ref[...], (((1,), (0,)), ((), ())),
                              preferred_element_type=jnp.int32)
    z = z32.astype(jnp.float32) * (
        (fscale_ref[0] / 127.0) * (1.0 / (127.0 * n)))
    x = jnp.maximum(
        jnp.dot(z, W1_ref[...], preferred_element_type=jnp.float32)
        + b1_ref[...], 0.0)
    x1_ref[...] = x.astype(jnp.bfloat16)
    m = jnp.full((1, 1), jnp.max(x), dtype=jnp.float32)

    @pl.when(j == 0)
    def _init_max():
        x1max_ref[...] = m

    @pl.when(j != 0)
    def _acc_max():
        x1max_ref[...] = jnp.maximum(x1max_ref[...], m)


def _layers2to5_body(adjq_ref, x1_ref, x1max_ref, Ws_ref, bs_ref, Wv1_ref,
                     bv1_ref, Wv2_ref, bv2_ref, out_ref, xq_ref, xbuf_ref,
                     smax_ref):
    l = pl.program_id(0)
    j = pl.program_id(1)
    blk = adjq_ref.shape[0]
    n = adjq_ref.shape[1]

    @pl.when(j == 0)
    def _requantize():
        @pl.when(l == 0)
        def _():
            smax_ref[0] = x1max_ref[0, 0]
        @pl.when(l != 0)
        def _():
            smax_ref[0] = smax_ref[1]
        scale = jnp.maximum(smax_ref[0], 1e-30)
        src = jnp.where(l == 0, x1_ref[...].astype(jnp.float32),
                        xbuf_ref[...].astype(jnp.float32))
        xq_ref[...] = (src * (127.0 / scale) + 0.5).astype(jnp.int8)
        smax_ref[1] = 0.0

    z32 = jax.lax.dot_general(adjq_ref[...], xq_ref[...],
                              (((1,), (0,)), ((), ())),
                              preferred_element_type=jnp.int32)
    scale_comb = (jnp.maximum(smax_ref[0], 1e-30) / 127.0) * (1.0 / (127.0 * n))
    z = z32.astype(jnp.float32) * scale_comb
    x_new = jnp.maximum(
        jnp.dot(z, Ws_ref[0], preferred_element_type=jnp.float32)
        + bs_ref[0], 0.0)
    xbuf_ref[pl.ds(j * blk, blk), :] = x_new.astype(jnp.bfloat16)
    smax_ref[1] = jnp.maximum(smax_ref[1], jnp.max(x_new))

    @pl.when(l == 3)
    def _value_head():
        h = jnp.maximum(
            jnp.dot(x_new, Wv1_ref[...], preferred_element_type=jnp.float32)
            + bv1_ref[...], 0.0)
        logit = (jnp.dot(h, Wv2_ref[...], preferred_element_type=jnp.float32)
                 + bv2_ref[...])
        out_ref[...] = jax.nn.sigmoid(logit)


def kernel(feat, adj, W1, b1, W2, b2, W3, b3, W4, b4, W5, b5, Wv1, bv1, Wv2,
           bv2):
    n, d = feat.shape
    blk = _pick_blk(n)
    nblk = n // blk
    blk2 = 1000 if n % 1000 == 0 else blk
    nblk2 = n // blk2

    Ws = jnp.stack([W2, W3, W4, W5])
    bs = jnp.stack([b2, b3, b4, b5]).reshape(4, 1, d)
    b1_2d = b1.reshape(1, d)
    bv1_2d = bv1.reshape(1, d)
    bv2_2d = bv2.reshape(1, 1)

    adjq, x1, x1max = pl.pallas_call(
        _layer1_body,
        grid=(nblk,),
        in_specs=[
            pl.BlockSpec((blk, n), lambda j: (j, 0)),
            pl.BlockSpec((n, d), lambda j: (0, 0)),
            pl.BlockSpec((d, d), lambda j: (0, 0)),
            pl.BlockSpec((1, d), lambda j: (0, 0)),
        ],
        out_specs=[
            pl.BlockSpec((blk, n), lambda j: (j, 0)),
            pl.BlockSpec((blk, d), lambda j: (j, 0)),
            pl.BlockSpec((1, 1), lambda j: (0, 0)),
        ],
        out_shape=[
            jax.ShapeDtypeStruct((n, n), jnp.int8),
            jax.ShapeDtypeStruct((n, d), jnp.bfloat16),
            jax.ShapeDtypeStruct((1, 1), jnp.float32),
        ],
        scratch_shapes=[
            pltpu.VMEM((n, d), jnp.int8),
            pltpu.SMEM((1,), jnp.float32),
        ],
        compiler_params=pltpu.CompilerParams(
            dimension_semantics=("arbitrary",)),
    )(adj, feat, W1, b1_2d)

    return pl.pallas_call(
        _layers2to5_body,
        grid=(4, nblk2),
        in_specs=[
            pl.BlockSpec((blk2, n), lambda l, j: (j, 0)),
            pl.BlockSpec((n, d), lambda l, j: (0, 0)),
            pl.BlockSpec(memory_space=pltpu.SMEM),
            pl.BlockSpec((1, d, d), lambda l, j: (l, 0, 0)),
            pl.BlockSpec((1, 1, d), lambda l, j: (l, 0, 0)),
            pl.BlockSpec((d, d), lambda l, j: (0, 0)),
            pl.BlockSpec((1, d), lambda l, j: (0, 0)),
            pl.BlockSpec((d, 1), lambda l, j: (0, 0)),
            pl.BlockSpec((1, 1), lambda l, j: (0, 0)),
        ],
        out_specs=pl.BlockSpec((blk2, 1), lambda l, j: (j, 0)),
        out_shape=jax.ShapeDtypeStruct((n, 1), jnp.float32),
        scratch_shapes=[
            pltpu.VMEM((n, d), jnp.int8),
            pltpu.VMEM((n, d), jnp.bfloat16),
            pltpu.SMEM((2,), jnp.float32),
        ],
        compiler_params=pltpu.CompilerParams(
            dimension_semantics=("arbitrary", "arbitrary")),
    )(adjq, x1, x1max, Ws, bs, Wv1, bv1_2d, Wv2, bv2_2d)


# speedup vs baseline: 1.1547x; 1.1547x over previous
import jax
import jax.numpy as jnp
from jax.experimental import pallas as pl
from jax.experimental.pallas import tpu as pltpu


def _pick_blk(n):
    for b in (400, 200, 100, 50, 25):
        if n % b == 0:
            return b
    return n


def _layer1_body(adjf_ref, feat_ref, W1_ref, b1_ref, adjq_ref, x1_ref,
                 x1max_ref, featq_ref, fscale_ref):
    j = pl.program_id(0)
    n = adjf_ref.shape[1]

    @pl.when(j == 0)
    def _quant_feat():
        f = feat_ref[...]
        fm = jnp.maximum(jnp.max(jnp.abs(f)), 1e-30)
        fscale_ref[0] = fm
        featq_ref[...] = (f * (256.0 / fm)).astype(jnp.float8_e4m3fn)

    q = (adjf_ref[...] * float(n)).astype(jnp.float8_e4m3fn)
    adjq_ref[...] = q
    z0 = jax.lax.dot_general(q, featq_ref[...], (((1,), (0,)), ((), ())),
                             preferred_element_type=jnp.float32)
    z = z0 * ((fscale_ref[0] / 256.0) * (1.0 / n))
    x = jnp.maximum(
        jnp.dot(z, W1_ref[...], preferred_element_type=jnp.float32)
        + b1_ref[...], 0.0)
    x1_ref[...] = x.astype(jnp.bfloat16)
    m = jnp.full((1, 1), jnp.max(x), dtype=jnp.float32)

    @pl.when(j == 0)
    def _init_max():
        x1max_ref[...] = m

    @pl.when(j != 0)
    def _acc_max():
        x1max_ref[...] = jnp.maximum(x1max_ref[...], m)


def _layers2to5_body(adjq_ref, x1_ref, x1max_ref, Ws_ref, bs_ref, Wv1_ref,
                     bv1_ref, Wv2_ref, bv2_ref, out_ref, xq_ref, xbuf_ref,
                     smax_ref):
    l = pl.program_id(0)
    j = pl.program_id(1)
    blk = adjq_ref.shape[0]
    n = adjq_ref.shape[1]

    @pl.when(j == 0)
    def _requantize():
        @pl.when(l == 0)
        def _():
            smax_ref[0] = x1max_ref[0, 0]
        @pl.when(l != 0)
        def _():
            smax_ref[0] = smax_ref[1]
        scale = jnp.maximum(smax_ref[0], 1e-30)
        src = jnp.where(l == 0, x1_ref[...].astype(jnp.float32),
                        xbuf_ref[...].astype(jnp.float32))
        xq_ref[...] = (src * (256.0 / scale)).astype(jnp.float8_e4m3fn)
        smax_ref[1] = 0.0

    z0 = jax.lax.dot_general(adjq_ref[...], xq_ref[...],
                             (((1,), (0,)), ((), ())),
                             preferred_element_type=jnp.float32)
    scale_comb = (jnp.maximum(smax_ref[0], 1e-30) / 256.0) * (1.0 / n)
    z = z0 * scale_comb
    x_new = jnp.maximum(
        jnp.dot(z, Ws_ref[0], preferred_element_type=jnp.float32)
        + bs_ref[0], 0.0)
    xbuf_ref[pl.ds(j * blk, blk), :] = x_new.astype(jnp.bfloat16)
    smax_ref[1] = jnp.maximum(smax_ref[1], jnp.max(x_new))

    @pl.when(l == 3)
    def _value_head():
        h = jnp.maximum(
            jnp.dot(x_new, Wv1_ref[...], preferred_element_type=jnp.float32)
            + bv1_ref[...], 0.0)
        logit = (jnp.dot(h, Wv2_ref[...], preferred_element_type=jnp.float32)
                 + bv2_ref[...])
        out_ref[...] = jax.nn.sigmoid(logit)


def kernel(feat, adj, W1, b1, W2, b2, W3, b3, W4, b4, W5, b5, Wv1, bv1, Wv2,
           bv2):
    n, d = feat.shape
    blk = _pick_blk(n)
    nblk = n // blk
    blk2 = 400 if n % 400 == 0 else blk
    nblk2 = n // blk2

    Ws = jnp.stack([W2, W3, W4, W5])
    bs = jnp.stack([b2, b3, b4, b5]).reshape(4, 1, d)
    b1_2d = b1.reshape(1, d)
    bv1_2d = bv1.reshape(1, d)
    bv2_2d = bv2.reshape(1, 1)

    adjq, x1, x1max = pl.pallas_call(
        _layer1_body,
        grid=(nblk,),
        in_specs=[
            pl.BlockSpec((blk, n), lambda j: (j, 0)),
            pl.BlockSpec((n, d), lambda j: (0, 0)),
            pl.BlockSpec((d, d), lambda j: (0, 0)),
            pl.BlockSpec((1, d), lambda j: (0, 0)),
        ],
        out_specs=[
            pl.BlockSpec((blk, n), lambda j: (j, 0)),
            pl.BlockSpec((blk, d), lambda j: (j, 0)),
            pl.BlockSpec((1, 1), lambda j: (0, 0)),
        ],
        out_shape=[
            jax.ShapeDtypeStruct((n, n), jnp.float8_e4m3fn),
            jax.ShapeDtypeStruct((n, d), jnp.bfloat16),
            jax.ShapeDtypeStruct((1, 1), jnp.float32),
        ],
        scratch_shapes=[
            pltpu.VMEM((n, d), jnp.float8_e4m3fn),
            pltpu.SMEM((1,), jnp.float32),
        ],
        compiler_params=pltpu.CompilerParams(
            dimension_semantics=("arbitrary",)),
    )(adj, feat, W1, b1_2d)

    return pl.pallas_call(
        _layers2to5_body,
        grid=(4, nblk2),
        in_specs=[
            pl.BlockSpec((blk2, n), lambda l, j: (j, 0)),
            pl.BlockSpec((n, d), lambda l, j: (0, 0)),
            pl.BlockSpec(memory_space=pltpu.SMEM),
            pl.BlockSpec((1, d, d), lambda l, j: (l, 0, 0)),
            pl.BlockSpec((1, 1, d), lambda l, j: (l, 0, 0)),
            pl.BlockSpec((d, d), lambda l, j: (0, 0)),
            pl.BlockSpec((1, d), lambda l, j: (0, 0)),
            pl.BlockSpec((d, 1), lambda l, j: (0, 0)),
            pl.BlockSpec((1, 1), lambda l, j: (0, 0)),
        ],
        out_specs=pl.BlockSpec((blk2, 1), lambda l, j: (j, 0)),
        out_shape=jax.ShapeDtypeStruct((n, 1), jnp.float32),
        scratch_shapes=[
            pltpu.VMEM((n, d), jnp.float8_e4m3fn),
            pltpu.VMEM((n, d), jnp.bfloat16),
            pltpu.SMEM((2,), jnp.float32),
        ],
        compiler_params=pltpu.CompilerParams(
            dimension_semantics=("arbitrary", "arbitrary")),
    )(adjq, x1, x1max, Ws, bs, Wv1, bv1_2d, Wv2, bv2_2d)


# speedup vs baseline: 1.2873x; 1.1149x over previous
import jax
import jax.numpy as jnp
from jax.experimental import pallas as pl
from jax.experimental.pallas import tpu as pltpu


def _pick_blk(n):
    for b in (400, 200, 100, 50, 25):
        if n % b == 0:
            return b
    return n


def _layer1_body(adjf_ref, feat_ref, W1_ref, b1_ref, adjq_ref, x1_ref,
                 x1max_ref, featq_ref, fscale_ref):
    j = pl.program_id(0)
    n = adjf_ref.shape[1]

    @pl.when(j == 0)
    def _quant_feat():
        f = feat_ref[...]
        fm = jnp.maximum(jnp.max(jnp.abs(f)), 1e-30)
        fscale_ref[0] = fm
        featq_ref[...] = (f * (256.0 / fm)).astype(jnp.float8_e4m3fn)

    q = (adjf_ref[...] * float(n)).astype(jnp.float8_e4m3fn)
    adjq_ref[...] = q
    z0 = jax.lax.dot_general(q, featq_ref[...], (((1,), (0,)), ((), ())),
                             preferred_element_type=jnp.float32)
    z = z0 * ((fscale_ref[0] / 256.0) * (1.0 / n))
    x = jnp.maximum(
        jnp.dot(z, W1_ref[...], preferred_element_type=jnp.float32)
        + b1_ref[...], 0.0)
    x1_ref[...] = x.astype(jnp.bfloat16)
    m = jnp.full((1, 1), jnp.max(x), dtype=jnp.float32)

    @pl.when(j == 0)
    def _init_max():
        x1max_ref[...] = m

    @pl.when(j != 0)
    def _acc_max():
        x1max_ref[...] = jnp.maximum(x1max_ref[...], m)


def _layers2to5_body(adjq_ref, x1_ref, x1max_ref, Ws_ref, bs_ref, Wv1_ref,
                     bv1_ref, Wv2_ref, bv2_ref, out_ref, xq_ref, xbuf_ref,
                     smax_ref):
    l = pl.program_id(0)
    j = pl.program_id(1)
    blk = adjq_ref.shape[0]
    n = adjq_ref.shape[1]

    @pl.when(j == 0)
    def _requantize():
        @pl.when(l == 0)
        def _():
            smax_ref[0] = x1max_ref[0, 0]
        @pl.when(l != 0)
        def _():
            smax_ref[0] = smax_ref[1]
        scale = jnp.maximum(smax_ref[0], 1e-30)
        src = jnp.where(l == 0, x1_ref[...].astype(jnp.float32),
                        xbuf_ref[...].astype(jnp.float32))
        xq_ref[...] = (src * (256.0 / scale)).astype(jnp.float8_e4m3fn)
        smax_ref[1] = 0.0

    z0 = jax.lax.dot_general(adjq_ref[...], xq_ref[...],
                             (((1,), (0,)), ((), ())),
                             preferred_element_type=jnp.float32)
    scale_comb = (jnp.maximum(smax_ref[0], 1e-30) / 256.0) * (1.0 / n)
    z = z0 * scale_comb
    x_new = jnp.maximum(
        jnp.dot(z, Ws_ref[0], preferred_element_type=jnp.float32)
        + bs_ref[0], 0.0)
    xbuf_ref[pl.ds(j * blk, blk), :] = x_new.astype(jnp.bfloat16)
    smax_ref[1] = jnp.maximum(smax_ref[1], jnp.max(x_new))

    @pl.when(l == 3)
    def _value_head():
        h = jnp.maximum(
            jnp.dot(x_new, Wv1_ref[...], preferred_element_type=jnp.float32)
            + bv1_ref[...], 0.0)
        logit = (jnp.dot(h, Wv2_ref[...], preferred_element_type=jnp.float32)
                 + bv2_ref[...])
        out_ref[...] = jax.nn.sigmoid(logit)


def kernel(feat, adj, W1, b1, W2, b2, W3, b3, W4, b4, W5, b5, Wv1, bv1, Wv2,
           bv2):
    n, d = feat.shape
    blk = _pick_blk(n)
    nblk = n // blk
    blk2 = 1000 if n % 1000 == 0 else blk
    nblk2 = n // blk2

    Ws = jnp.stack([W2, W3, W4, W5])
    bs = jnp.stack([b2, b3, b4, b5]).reshape(4, 1, d)
    b1_2d = b1.reshape(1, d)
    bv1_2d = bv1.reshape(1, d)
    bv2_2d = bv2.reshape(1, 1)

    adjq, x1, x1max = pl.pallas_call(
        _layer1_body,
        grid=(nblk,),
        in_specs=[
            pl.BlockSpec((blk, n), lambda j: (j, 0)),
            pl.BlockSpec((n, d), lambda j: (0, 0)),
            pl.BlockSpec((d, d), lambda j: (0, 0)),
            pl.BlockSpec((1, d), lambda j: (0, 0)),
        ],
        out_specs=[
            pl.BlockSpec((blk, n), lambda j: (j, 0)),
            pl.BlockSpec((blk, d), lambda j: (j, 0)),
            pl.BlockSpec((1, 1), lambda j: (0, 0)),
        ],
        out_shape=[
            jax.ShapeDtypeStruct((n, n), jnp.float8_e4m3fn),
            jax.ShapeDtypeStruct((n, d), jnp.bfloat16),
            jax.ShapeDtypeStruct((1, 1), jnp.float32),
        ],
        scratch_shapes=[
            pltpu.VMEM((n, d), jnp.float8_e4m3fn),
            pltpu.SMEM((1,), jnp.float32),
        ],
        compiler_params=pltpu.CompilerParams(
            dimension_semantics=("arbitrary",)),
    )(adj, feat, W1, b1_2d)

    return pl.pallas_call(
        _layers2to5_body,
        grid=(4, nblk2),
        in_specs=[
            pl.BlockSpec((blk2, n), lambda l, j: (j, 0)),
            pl.BlockSpec((n, d), lambda l, j: (0, 0)),
            pl.BlockSpec(memory_space=pltpu.SMEM),
            pl.BlockSpec((1, d, d), lambda l, j: (l, 0, 0)),
            pl.BlockSpec((1, 1, d), lambda l, j: (l, 0, 0)),
            pl.BlockSpec((d, d), lambda l, j: (0, 0)),
            pl.BlockSpec((1, d), lambda l, j: (0, 0)),
            pl.BlockSpec((d, 1), lambda l, j: (0, 0)),
            pl.BlockSpec((1, 1), lambda l, j: (0, 0)),
        ],
        out_specs=pl.BlockSpec((blk2, 1), lambda l, j: (j, 0)),
        out_shape=jax.ShapeDtypeStruct((n, 1), jnp.float32),
        scratch_shapes=[
            pltpu.VMEM((n, d), jnp.float8_e4m3fn),
            pltpu.VMEM((n, d), jnp.bfloat16),
            pltpu.SMEM((2,), jnp.float32),
        ],
        compiler_params=pltpu.CompilerParams(
            dimension_semantics=("arbitrary", "arbitrary")),
    )(adjq, x1, x1max, Ws, bs, Wv1, bv1_2d, Wv2, bv2_2d)
